# baseline (speedup 1.0000x reference)
import jax
import jax.numpy as jnp
from jax.experimental import pallas as pl
from jax.experimental.pallas import tpu as pltpu


_LANES = 1024
_SAMPLES_PER_BLOCK = 2


def _add_noise_block(t_ref, sa_ref, sb_ref, x0_ref, noise_ref, out_ref):
    i = pl.program_id(0)
    rows = x0_ref.shape[0] // _SAMPLES_PER_BLOCK
    for k in range(_SAMPLES_PER_BLOCK):
        tt = t_ref[i * _SAMPLES_PER_BLOCK + k]
        a = sa_ref[tt]
        b = sb_ref[tt]
        sl = pl.ds(k * rows, rows)
        out_ref[sl, :] = a * x0_ref[sl, :] + b * noise_ref[sl, :]


def kernel(x0, noise, t, sqrt_alphas_cumprod, sqrt_one_minus_alphas_cumprod):
    n, c, h, w = x0.shape
    rows = c * h * w // _LANES
    x2 = x0.reshape(n * rows, _LANES)
    n2 = noise.reshape(n * rows, _LANES)
    blk_rows = rows * _SAMPLES_PER_BLOCK

    out = pl.pallas_call(
        _add_noise_block,
        grid_spec=pltpu.PrefetchScalarGridSpec(
            num_scalar_prefetch=3,
            grid=(n // _SAMPLES_PER_BLOCK,),
            in_specs=[
                pl.BlockSpec((blk_rows, _LANES), lambda i, *_: (i, 0)),
                pl.BlockSpec((blk_rows, _LANES), lambda i, *_: (i, 0)),
            ],
            out_specs=pl.BlockSpec((blk_rows, _LANES), lambda i, *_: (i, 0)),
        ),
        out_shape=jax.ShapeDtypeStruct((n * rows, _LANES), x0.dtype),
        compiler_params=pltpu.CompilerParams(
            dimension_semantics=("arbitrary",),
        ),
    )(t, sqrt_alphas_cumprod, sqrt_one_minus_alphas_cumprod, x2, n2)
    return out.reshape(n, c, h, w)


# speedup vs baseline: 4.4451x; 4.4451x over previous
import jax
import jax.numpy as jnp
from jax.experimental import pallas as pl
from jax.experimental.pallas import tpu as pltpu


_LANES = 256
_SAMPLES_PER_BLOCK = 4


def _add_noise_block(t_ref, sa_ref, sb_ref, x0_ref, noise_ref, out_ref):
    i = pl.program_id(0)
    rows = x0_ref.shape[0] // _SAMPLES_PER_BLOCK
    for k in range(_SAMPLES_PER_BLOCK):
        tt = t_ref[i * _SAMPLES_PER_BLOCK + k]
        a = sa_ref[tt]
        b = sb_ref[tt]
        sl = pl.ds(k * rows, rows)
        out_ref[sl, :] = a * x0_ref[sl, :] + b * noise_ref[sl, :]


def kernel(x0, noise, t, sqrt_alphas_cumprod, sqrt_one_minus_alphas_cumprod):
    n, c, h, w = x0.shape
    rows = c * h * w // _LANES
    x2 = x0.reshape(n * rows, _LANES)
    n2 = noise.reshape(n * rows, _LANES)
    blk_rows = rows * _SAMPLES_PER_BLOCK

    out = pl.pallas_call(
        _add_noise_block,
        grid_spec=pltpu.PrefetchScalarGridSpec(
            num_scalar_prefetch=3,
            grid=(n // _SAMPLES_PER_BLOCK,),
            in_specs=[
                pl.BlockSpec((blk_rows, _LANES), lambda i, *_: (i, 0)),
                pl.BlockSpec((blk_rows, _LANES), lambda i, *_: (i, 0)),
            ],
            out_specs=pl.BlockSpec((blk_rows, _LANES), lambda i, *_: (i, 0)),
        ),
        out_shape=jax.ShapeDtypeStruct((n * rows, _LANES), x0.dtype),
        compiler_params=pltpu.CompilerParams(
            dimension_semantics=("arbitrary",),
        ),
    )(t, sqrt_alphas_cumprod, sqrt_one_minus_alphas_cumprod, x2, n2)
    return out.reshape(n, c, h, w)
